# initial kernel scaffold (unmeasured)
import jax
import jax.numpy as jnp
from jax import lax
from jax.experimental import pallas as pl
from jax.experimental.pallas import tpu as pltpu


def kernel(
    x,
):
    def body(*refs):
        pass

    out_shape = jax.ShapeDtypeStruct(..., jnp.float32)
    return pl.pallas_call(body, out_shape=out_shape)(...)


# baseline (device time: 818164 ns/iter reference)
import jax
import jax.numpy as jnp
from jax import lax
from jax.experimental import pallas as pl
from jax.experimental.pallas import tpu as pltpu

M, N = 16384, 1024
N_CHUNKS = 8
MC = M // N_CHUNKS


def kernel(x):
    def body(x_hbm, out_hbm, my_buf, recv_buf, local_sems, out_sems,
             send_sems, recv_sems):
        my_x = lax.axis_index("x")
        my_y = lax.axis_index("y")
        nbr = (my_x, 1 - my_y)

        barrier_sem = pltpu.get_barrier_semaphore()
        pl.semaphore_signal(
            barrier_sem, inc=1, device_id=nbr,
            device_id_type=pl.DeviceIdType.MESH,
        )
        pl.semaphore_wait(barrier_sem, 1)

        for c in range(N_CHUNKS):
            slot = c % 2
            rows = pl.ds(c * MC, MC)

            rdma = pltpu.make_async_remote_copy(
                src_ref=x_hbm.at[rows, :],
                dst_ref=recv_buf.at[slot],
                send_sem=send_sems.at[slot],
                recv_sem=recv_sems.at[slot],
                device_id=nbr,
                device_id_type=pl.DeviceIdType.MESH,
            )
            rdma.start()

            cp_in = pltpu.make_async_copy(
                x_hbm.at[rows, :], my_buf.at[slot], local_sems.at[slot]
            )
            cp_in.start()
            cp_in.wait()
            rdma.wait()

            my_buf[slot] = my_buf[slot] + recv_buf[slot]

            cp_out = pltpu.make_async_copy(
                my_buf.at[slot], out_hbm.at[rows, :], out_sems.at[slot]
            )
            cp_out.start()
            cp_out.wait()

    out_shape = jax.ShapeDtypeStruct((M, N), jnp.float32)
    return pl.pallas_call(
        body,
        out_shape=out_shape,
        in_specs=[pl.BlockSpec(memory_space=pl.ANY)],
        out_specs=pl.BlockSpec(memory_space=pl.ANY),
        scratch_shapes=[
            pltpu.VMEM((2, MC, N), jnp.float32),
            pltpu.VMEM((2, MC, N), jnp.float32),
            pltpu.SemaphoreType.DMA((2,)),
            pltpu.SemaphoreType.DMA((2,)),
            pltpu.SemaphoreType.DMA((2,)),
            pltpu.SemaphoreType.DMA((2,)),
        ],
        compiler_params=pltpu.CompilerParams(collective_id=0),
    )(x)


# device time: 471413 ns/iter; 1.7356x vs baseline; 1.7356x over previous
import jax
import jax.numpy as jnp
from jax import lax
from jax.experimental import pallas as pl
from jax.experimental.pallas import tpu as pltpu

M, N = 16384, 1024
HALF = M // 2
MC = 512
NC = HALF // MC


def kernel(x):
    def body(x_hbm, out_hbm, my_buf, yrecv_buf, sum_buf, local_sems,
             out_sems, y_send_sems, y_recv_sems, x_send_sems, x_recv_sems):
        my_x = lax.axis_index("x")
        my_y = lax.axis_index("y")
        nbr_y = (my_x, 1 - my_y)
        nbr_x = (1 - my_x, my_y)
        base = my_x * HALF

        barrier_sem = pltpu.get_barrier_semaphore()
        for nbr in (nbr_y, nbr_x):
            pl.semaphore_signal(
                barrier_sem, inc=1, device_id=nbr,
                device_id_type=pl.DeviceIdType.MESH,
            )
        pl.semaphore_wait(barrier_sem, 2)

        def rows(c):
            return pl.ds(base + c * MC, MC)

        def make_y(c):
            s = c % 2
            return pltpu.make_async_remote_copy(
                src_ref=x_hbm.at[rows(c), :],
                dst_ref=yrecv_buf.at[s],
                send_sem=y_send_sems.at[s],
                recv_sem=y_recv_sems.at[s],
                device_id=nbr_y,
                device_id_type=pl.DeviceIdType.MESH,
            )

        def make_x(c):
            s = c % 2
            return pltpu.make_async_remote_copy(
                src_ref=sum_buf.at[s],
                dst_ref=out_hbm.at[rows(c), :],
                send_sem=x_send_sems.at[s],
                recv_sem=x_recv_sems.at[s],
                device_id=nbr_x,
                device_id_type=pl.DeviceIdType.MESH,
            )

        def make_local(c):
            s = c % 2
            return pltpu.make_async_copy(
                x_hbm.at[rows(c), :], my_buf.at[s], local_sems.at[s]
            )

        y_rdma, x_rdma, loc, cp_out = {}, {}, {}, {}

        y_rdma[0] = make_y(0)
        y_rdma[0].start()
        loc[0] = make_local(0)
        loc[0].start()

        for c in range(NC):
            s = c % 2
            loc[c].wait()
            y_rdma[c].wait_recv()
            y_rdma[c].wait_send()
            if c + 1 < NC:
                y_rdma[c + 1] = make_y(c + 1)
                y_rdma[c + 1].start()
                loc[c + 1] = make_local(c + 1)
                loc[c + 1].start()
            if c >= 1:
                x_rdma[c - 1].wait_recv()
                x_rdma[c - 1].wait_send()
            if c >= 2:
                cp_out[c - 2].wait()
            sum_buf[s] = my_buf[s] + yrecv_buf[s]
            x_rdma[c] = make_x(c)
            x_rdma[c].start()
            cp_out[c] = pltpu.make_async_copy(
                sum_buf.at[s], out_hbm.at[rows(c), :], out_sems.at[s]
            )
            cp_out[c].start()

        x_rdma[NC - 1].wait_recv()
        x_rdma[NC - 1].wait_send()
        cp_out[NC - 2].wait()
        cp_out[NC - 1].wait()

    out_shape = jax.ShapeDtypeStruct((M, N), jnp.float32)
    return pl.pallas_call(
        body,
        out_shape=out_shape,
        in_specs=[pl.BlockSpec(memory_space=pl.ANY)],
        out_specs=pl.BlockSpec(memory_space=pl.ANY),
        scratch_shapes=[
            pltpu.VMEM((2, MC, N), jnp.float32),
            pltpu.VMEM((2, MC, N), jnp.float32),
            pltpu.VMEM((2, MC, N), jnp.float32),
            pltpu.SemaphoreType.DMA((2,)),
            pltpu.SemaphoreType.DMA((2,)),
            pltpu.SemaphoreType.DMA((2,)),
            pltpu.SemaphoreType.DMA((2,)),
            pltpu.SemaphoreType.DMA((2,)),
            pltpu.SemaphoreType.DMA((2,)),
        ],
        compiler_params=pltpu.CompilerParams(collective_id=0),
    )(x)


# device time: 433460 ns/iter; 1.8875x vs baseline; 1.0876x over previous
import jax
import jax.numpy as jnp
from jax import lax
from jax.experimental import pallas as pl
from jax.experimental.pallas import tpu as pltpu

M, N = 16384, 1024
HALF = M // 2
MC = 512
NC = HALF // MC
S = 4


def kernel(x):
    def body(x_hbm, out_hbm, my_buf, yrecv_buf, sum_buf, local_sems,
             out_sems, y_send_sems, y_recv_sems, x_send_sems, x_recv_sems):
        my_x = lax.axis_index("x")
        my_y = lax.axis_index("y")
        nbr_y = (my_x, 1 - my_y)
        nbr_x = (1 - my_x, my_y)
        base = my_x * HALF

        barrier_sem = pltpu.get_barrier_semaphore()
        for nbr in (nbr_y, nbr_x):
            pl.semaphore_signal(
                barrier_sem, inc=1, device_id=nbr,
                device_id_type=pl.DeviceIdType.MESH,
            )
        pl.semaphore_wait(barrier_sem, 2)

        def rows(c):
            return pl.ds(base + c * MC, MC)

        def make_y(c):
            s = c % S
            return pltpu.make_async_remote_copy(
                src_ref=x_hbm.at[rows(c), :],
                dst_ref=yrecv_buf.at[s],
                send_sem=y_send_sems.at[s],
                recv_sem=y_recv_sems.at[s],
                device_id=nbr_y,
                device_id_type=pl.DeviceIdType.MESH,
            )

        def make_x(c):
            s = c % S
            return pltpu.make_async_remote_copy(
                src_ref=sum_buf.at[s],
                dst_ref=out_hbm.at[rows(c), :],
                send_sem=x_send_sems.at[s],
                recv_sem=x_recv_sems.at[s],
                device_id=nbr_x,
                device_id_type=pl.DeviceIdType.MESH,
            )

        def make_local(c):
            s = c % S
            return pltpu.make_async_copy(
                x_hbm.at[rows(c), :], my_buf.at[s], local_sems.at[s]
            )

        y_rdma, x_rdma, loc, cp_out = {}, {}, {}, {}

        for c in (0, 1):
            y_rdma[c] = make_y(c)
            y_rdma[c].start()
            loc[c] = make_local(c)
            loc[c].start()

        for c in range(NC):
            s = c % S
            loc[c].wait()
            y_rdma[c].wait_recv()
            y_rdma[c].wait_send()
            if c + 2 < NC:
                y_rdma[c + 2] = make_y(c + 2)
                y_rdma[c + 2].start()
                loc[c + 2] = make_local(c + 2)
                loc[c + 2].start()
            if c >= 2:
                x_rdma[c - 2].wait_recv()
                x_rdma[c - 2].wait_send()
            if c >= S:
                cp_out[c - S].wait()
            sum_buf[s] = my_buf[s] + yrecv_buf[s]
            x_rdma[c] = make_x(c)
            x_rdma[c].start()
            cp_out[c] = pltpu.make_async_copy(
                sum_buf.at[s], out_hbm.at[rows(c), :], out_sems.at[s]
            )
            cp_out[c].start()

        for c in (NC - 2, NC - 1):
            x_rdma[c].wait_recv()
            x_rdma[c].wait_send()
        for c in range(NC - S, NC):
            cp_out[c].wait()

    out_shape = jax.ShapeDtypeStruct((M, N), jnp.float32)
    return pl.pallas_call(
        body,
        out_shape=out_shape,
        in_specs=[pl.BlockSpec(memory_space=pl.ANY)],
        out_specs=pl.BlockSpec(memory_space=pl.ANY),
        scratch_shapes=[
            pltpu.VMEM((S, MC, N), jnp.float32),
            pltpu.VMEM((S, MC, N), jnp.float32),
            pltpu.VMEM((S, MC, N), jnp.float32),
            pltpu.SemaphoreType.DMA((S,)),
            pltpu.SemaphoreType.DMA((S,)),
            pltpu.SemaphoreType.DMA((S,)),
            pltpu.SemaphoreType.DMA((S,)),
            pltpu.SemaphoreType.DMA((S,)),
            pltpu.SemaphoreType.DMA((S,)),
        ],
        compiler_params=pltpu.CompilerParams(collective_id=0),
    )(x)
